# baseline (speedup 1.0000x reference)
import jax
import jax.numpy as jnp
from jax import lax
from jax.experimental import pallas as pl
from jax.experimental.pallas import tpu as pltpu
from jax.experimental.pallas import tpu_sc as plsc

N_NODES = 10000
NPAD = 10240
DIM = 128
N_EDGES = 320000
NUM_HEADS = 8

NC = 2
NS = 16
NW = NC * NS
CHUNK = 128
NSLICE = NW
SLICE_E = 10240
E_PAD = NSLICE * SLICE_E
NSUPER = 8
SUBCH = SLICE_E // (NSUPER * CHUNK)
ROWS_PER_TILE = NPAD // NS
DUMMY_DST = 10200


def _gather_scatter_ring(feat_hbm, src_v, dst_v, acc_sh, rows_v, rows_w,
                         gsem0, gsem1):
    def chunk_body(c, carry):
        pltpu.async_copy(feat_hbm.at[src_v.at[c]], rows_v, gsem0).wait()
        pltpu.sync_copy(rows_v, acc_sh.at[dst_v.at[c]], add=True)
        return carry

    lax.fori_loop(0, SUBCH, chunk_body, 0)


def _sc_layer1(feat, src3, dst3, zeros_nd, ones_rows):
    def body(feat_hbm, src_hbm, dst_hbm, zeros_hbm, ones_hbm, out_hbm,
             src_v, dst_v, rows_v, rows_w, acc_sh, gsem0, gsem1):
        cid = lax.axis_index("c")
        sid = lax.axis_index("s")
        row0 = sid * ROWS_PER_TILE
        pltpu.sync_copy(zeros_hbm.at[pl.ds(row0, ROWS_PER_TILE)],
                        acc_sh.at[pl.ds(row0, ROWS_PER_TILE)])

        @pl.when(cid == 1)
        def _():
            pltpu.sync_copy(ones_hbm, rows_v)

        plsc.subcore_barrier()

        @pl.when(cid == 0)
        def _():
            def run_slice(t, carry):
                w = sid * 2 + t

                def super_body(s, carry1):
                    pltpu.sync_copy(dst_hbm.at[w, s], dst_v)
                    pltpu.sync_copy(src_hbm.at[w, s], src_v)
                    _gather_scatter_ring(feat_hbm, src_v, dst_v, acc_sh,
                                         rows_v, rows_w, gsem0, gsem1)
                    return carry1

                lax.fori_loop(0, NSUPER, super_body, 0)
                return carry

            lax.fori_loop(0, 2, run_slice, 0)

        @pl.when(cid == 1)
        def _():
            def run_slice(t, carry):
                w = sid * 2 + t

                def super_body(s, carry1):
                    pltpu.sync_copy(dst_hbm.at[w, s], dst_v)

                    def chunk_body(c, carry2):
                        pltpu.sync_copy(rows_v, acc_sh.at[dst_v.at[c]],
                                        add=True)
                        return carry2

                    lax.fori_loop(0, SUBCH, chunk_body, 0)
                    return carry1

                lax.fori_loop(0, NSUPER, super_body, 0)
                return carry

            lax.fori_loop(0, 2, run_slice, 0)

        plsc.subcore_barrier()

        pltpu.sync_copy(acc_sh.at[pl.ds(row0, ROWS_PER_TILE)],
                        out_hbm.at[cid, pl.ds(row0, ROWS_PER_TILE)])

    mesh = plsc.VectorSubcoreMesh(core_axis_name="c", subcore_axis_name="s",
                                  num_cores=NC, num_subcores=NS)
    return pl.kernel(
        body,
        out_type=jax.ShapeDtypeStruct((NC, NPAD, DIM), jnp.float32),
        mesh=mesh,
        scratch_types=[
            pltpu.VMEM((SUBCH, CHUNK), jnp.int32),
            pltpu.VMEM((SUBCH, CHUNK), jnp.int32),
            pltpu.VMEM((CHUNK, DIM), jnp.float32),
            pltpu.VMEM((CHUNK, DIM), jnp.float32),
            pltpu.VMEM_SHARED((NPAD, DIM), jnp.float32),
            pltpu.SemaphoreType.DMA,
            pltpu.SemaphoreType.DMA,
        ],
        name="sc_sum_and_deg",
    )(feat, src3, dst3, zeros_nd, ones_rows)


def _sc_layer2(feat, src3, dst3, zeros_nd):
    def body(feat_hbm, src_hbm, dst_hbm, zeros_hbm, out_hbm,
             src_v, dst_v, rows_v, rows_w, acc_sh, gsem0, gsem1):
        cid = lax.axis_index("c")
        sid = lax.axis_index("s")
        wid = cid * NS + sid
        row0 = sid * ROWS_PER_TILE
        pltpu.sync_copy(zeros_hbm.at[pl.ds(row0, ROWS_PER_TILE)],
                        acc_sh.at[pl.ds(row0, ROWS_PER_TILE)])
        plsc.subcore_barrier()

        def super_body(s, carry):
            pltpu.sync_copy(src_hbm.at[wid, s], src_v)
            pltpu.sync_copy(dst_hbm.at[wid, s], dst_v)
            _gather_scatter_ring(feat_hbm, src_v, dst_v, acc_sh,
                                 rows_v, rows_w, gsem0, gsem1)
            return carry

        lax.fori_loop(0, NSUPER, super_body, 0)
        plsc.subcore_barrier()

        pltpu.sync_copy(acc_sh.at[pl.ds(row0, ROWS_PER_TILE)],
                        out_hbm.at[cid, pl.ds(row0, ROWS_PER_TILE)])

    mesh = plsc.VectorSubcoreMesh(core_axis_name="c", subcore_axis_name="s",
                                  num_cores=NC, num_subcores=NS)
    return pl.kernel(
        body,
        out_type=jax.ShapeDtypeStruct((NC, NPAD, DIM), jnp.float32),
        mesh=mesh,
        scratch_types=[
            pltpu.VMEM((SUBCH, CHUNK), jnp.int32),
            pltpu.VMEM((SUBCH, CHUNK), jnp.int32),
            pltpu.VMEM((CHUNK, DIM), jnp.float32),
            pltpu.VMEM((CHUNK, DIM), jnp.float32),
            pltpu.VMEM_SHARED((NPAD, DIM), jnp.float32),
            pltpu.SemaphoreType.DMA,
            pltpu.SemaphoreType.DMA,
        ],
        name="sc_seg_sum",
    )(feat, src3, dst3, zeros_nd)


ROW_BLK = 1000
N_BLKS = N_NODES // ROW_BLK


def _tc_sage_body(x_ref, p_ref, deg_ref, wself_ref, wneigh_ref, b_ref, out_ref):
    deg = jnp.maximum(deg_ref[0], 1.0)
    psum = p_ref[0]
    for c in range(1, p_ref.shape[0]):
        psum = psum + p_ref[c]
    agg = psum / deg
    acc = jnp.dot(x_ref[...], wself_ref[...], preferred_element_type=jnp.float32)
    acc += jnp.dot(agg, wneigh_ref[...], preferred_element_type=jnp.float32)
    out_ref[...] = jnp.maximum(acc + b_ref[...], 0.0)


def _tc_sage(x, partials, npart, deg_arr, w_self, w_neigh, b):
    out_dim = w_self.shape[1]
    return pl.pallas_call(
        _tc_sage_body,
        grid=(N_BLKS,),
        in_specs=[
            pl.BlockSpec((ROW_BLK, DIM), lambda i: (i, 0)),
            pl.BlockSpec((npart, ROW_BLK, DIM), lambda i: (0, i, 0)),
            pl.BlockSpec((1, ROW_BLK, DIM), lambda i: (1, i, 0)),
            pl.BlockSpec((DIM, out_dim), lambda i: (0, 0)),
            pl.BlockSpec((DIM, out_dim), lambda i: (0, 0)),
            pl.BlockSpec((1, out_dim), lambda i: (0, 0)),
        ],
        out_specs=pl.BlockSpec((ROW_BLK, out_dim), lambda i: (i, 0)),
        out_shape=jax.ShapeDtypeStruct((N_NODES, out_dim), jnp.float32),
    )(x, partials, deg_arr, w_self, w_neigh, b)


def _pad_edges(idx_row, fill):
    pad = jnp.full((E_PAD - N_EDGES,), fill, jnp.int32)
    return jnp.concatenate([idx_row.astype(jnp.int32), pad]).reshape(
        NSLICE, NSUPER, SUBCH, CHUNK)


def kernel(x, edge_index, Ws_self, Ws_neigh, bs, Wh_self, Wh_neigh, bh):
    src = _pad_edges(edge_index[0], 0)
    dst = _pad_edges(edge_index[1], DUMMY_DST)
    zeros_nd = jnp.zeros((NPAD, DIM), jnp.float32)
    ones_rows = jnp.ones((CHUNK, DIM), jnp.float32)

    out1 = _sc_layer1(x, src, dst, zeros_nd, ones_rows)
    shared = _tc_sage(x, out1, 1, out1, Ws_self, Ws_neigh, bs.reshape(1, DIM))

    p2 = _sc_layer2(shared, src, dst, zeros_nd)
    w_self_cat = jnp.transpose(Wh_self, (1, 0, 2)).reshape(DIM, NUM_HEADS * DIM)
    w_neigh_cat = jnp.transpose(Wh_neigh, (1, 0, 2)).reshape(DIM, NUM_HEADS * DIM)
    b_cat = bh.reshape(1, NUM_HEADS * DIM)
    heads = _tc_sage(shared, p2, 2, out1, w_self_cat, w_neigh_cat, b_cat)
    return heads.reshape(N_NODES, NUM_HEADS, DIM)


# speedup vs baseline: 2.2255x; 2.2255x over previous
import jax
import jax.numpy as jnp
from jax import lax
from jax.experimental import pallas as pl
from jax.experimental.pallas import tpu as pltpu
from jax.experimental.pallas import tpu_sc as plsc

N_NODES = 10000
NPAD = 10240
DIM = 128
N_EDGES = 320000
NUM_HEADS = 8

NC = 2
NS = 16
NW = NC * NS
CHUNK = 80
NSLICE = NW
SLICE_E = N_EDGES // NSLICE
NSUPER = 5
SUBCH = SLICE_E // (NSUPER * CHUNK)
ROWS_PER_TILE = NPAD // NS


def _gather_scatter_ring(feat_hbm, src_v, dst_v, acc_sh, rows_v, rows_w,
                         gsem0, gsem1):
    pltpu.async_copy(feat_hbm.at[src_v.at[0]], rows_v, gsem0)

    def chunk_body(c, carry):
        for par in range(2):
            buf, sem = (rows_v, gsem0) if par == 0 else (rows_w, gsem1)
            obuf, osem = (rows_w, gsem1) if par == 0 else (rows_v, gsem0)
            cc = c * 2 + par
            pltpu.make_async_copy(feat_hbm.at[src_v.at[cc]], buf, sem).wait()

            @pl.when(cc + 1 < SUBCH)
            def _():
                pltpu.async_copy(feat_hbm.at[src_v.at[cc + 1]], obuf, osem)

            pltpu.sync_copy(buf, acc_sh.at[dst_v.at[cc]], add=True)
        return carry

    lax.fori_loop(0, SUBCH // 2, chunk_body, 0)
    if SUBCH % 2:
        cc = SUBCH - 1
        pltpu.make_async_copy(feat_hbm.at[src_v.at[cc]], rows_v, gsem0).wait()
        pltpu.sync_copy(rows_v, acc_sh.at[dst_v.at[cc]], add=True)


def _sc_layer1(feat, src3, dst3, zeros_nd, ones_rows):
    def body(feat_hbm, src_hbm, dst_hbm, zeros_hbm, ones_hbm, out_hbm,
             src_v, dst_v, rows_v, rows_w, acc_sh, gsem0, gsem1):
        cid = lax.axis_index("c")
        sid = lax.axis_index("s")
        row0 = sid * ROWS_PER_TILE
        pltpu.sync_copy(zeros_hbm.at[pl.ds(row0, ROWS_PER_TILE)],
                        acc_sh.at[pl.ds(row0, ROWS_PER_TILE)])

        @pl.when(cid == 1)
        def _():
            pltpu.sync_copy(ones_hbm, rows_v)

        plsc.subcore_barrier()

        @pl.when(cid == 0)
        def _():
            def run_slice(t, carry):
                w = sid * 2 + t

                def super_body(s, carry1):
                    pltpu.sync_copy(dst_hbm.at[w, s], dst_v)
                    pltpu.sync_copy(src_hbm.at[w, s], src_v)
                    _gather_scatter_ring(feat_hbm, src_v, dst_v, acc_sh,
                                         rows_v, rows_w, gsem0, gsem1)
                    return carry1

                lax.fori_loop(0, NSUPER, super_body, 0)
                return carry

            lax.fori_loop(0, 2, run_slice, 0)

        @pl.when(cid == 1)
        def _():
            def run_slice(t, carry):
                w = sid * 2 + t

                def super_body(s, carry1):
                    pltpu.sync_copy(dst_hbm.at[w, s], dst_v)

                    def chunk_body(c, carry2):
                        pltpu.sync_copy(rows_v, acc_sh.at[dst_v.at[c]],
                                        add=True)
                        return carry2

                    lax.fori_loop(0, SUBCH, chunk_body, 0)
                    return carry1

                lax.fori_loop(0, NSUPER, super_body, 0)
                return carry

            lax.fori_loop(0, 2, run_slice, 0)

        plsc.subcore_barrier()

        pltpu.sync_copy(acc_sh.at[pl.ds(row0, ROWS_PER_TILE)],
                        out_hbm.at[cid, pl.ds(row0, ROWS_PER_TILE)])

    mesh = plsc.VectorSubcoreMesh(core_axis_name="c", subcore_axis_name="s",
                                  num_cores=NC, num_subcores=NS)
    return pl.kernel(
        body,
        out_type=jax.ShapeDtypeStruct((NC, NPAD, DIM), jnp.float32),
        mesh=mesh,
        scratch_types=[
            pltpu.VMEM((SUBCH, CHUNK), jnp.int32),
            pltpu.VMEM((SUBCH, CHUNK), jnp.int32),
            pltpu.VMEM((CHUNK, DIM), jnp.float32),
            pltpu.VMEM((CHUNK, DIM), jnp.float32),
            pltpu.VMEM_SHARED((NPAD, DIM), jnp.float32),
            pltpu.SemaphoreType.DMA,
            pltpu.SemaphoreType.DMA,
        ],
        name="sc_sum_and_deg",
    )(feat, src3, dst3, zeros_nd, ones_rows)


def _sc_layer2(feat, src3, dst3, zeros_nd):
    def body(feat_hbm, src_hbm, dst_hbm, zeros_hbm, out_hbm,
             src_v, dst_v, rows_v, rows_w, acc_sh, gsem0, gsem1):
        cid = lax.axis_index("c")
        sid = lax.axis_index("s")
        wid = cid * NS + sid
        row0 = sid * ROWS_PER_TILE
        pltpu.sync_copy(zeros_hbm.at[pl.ds(row0, ROWS_PER_TILE)],
                        acc_sh.at[pl.ds(row0, ROWS_PER_TILE)])
        plsc.subcore_barrier()

        def super_body(s, carry):
            pltpu.sync_copy(src_hbm.at[wid, s], src_v)
            pltpu.sync_copy(dst_hbm.at[wid, s], dst_v)
            _gather_scatter_ring(feat_hbm, src_v, dst_v, acc_sh,
                                 rows_v, rows_w, gsem0, gsem1)
            return carry

        lax.fori_loop(0, NSUPER, super_body, 0)
        plsc.subcore_barrier()

        pltpu.sync_copy(acc_sh.at[pl.ds(row0, ROWS_PER_TILE)],
                        out_hbm.at[cid, pl.ds(row0, ROWS_PER_TILE)])

    mesh = plsc.VectorSubcoreMesh(core_axis_name="c", subcore_axis_name="s",
                                  num_cores=NC, num_subcores=NS)
    return pl.kernel(
        body,
        out_type=jax.ShapeDtypeStruct((NC, NPAD, DIM), jnp.float32),
        mesh=mesh,
        scratch_types=[
            pltpu.VMEM((SUBCH, CHUNK), jnp.int32),
            pltpu.VMEM((SUBCH, CHUNK), jnp.int32),
            pltpu.VMEM((CHUNK, DIM), jnp.float32),
            pltpu.VMEM((CHUNK, DIM), jnp.float32),
            pltpu.VMEM_SHARED((NPAD, DIM), jnp.float32),
            pltpu.SemaphoreType.DMA,
            pltpu.SemaphoreType.DMA,
        ],
        name="sc_seg_sum",
    )(feat, src3, dst3, zeros_nd)


ROW_BLK = 1000
N_BLKS = N_NODES // ROW_BLK


def _tc_sage_body(x_ref, p_ref, deg_ref, wself_ref, wneigh_ref, b_ref, out_ref):
    deg = jnp.maximum(deg_ref[0], 1.0)
    psum = p_ref[0]
    for c in range(1, p_ref.shape[0]):
        psum = psum + p_ref[c]
    agg = psum / deg
    acc = jnp.dot(x_ref[...], wself_ref[...], preferred_element_type=jnp.float32)
    acc += jnp.dot(agg, wneigh_ref[...], preferred_element_type=jnp.float32)
    out_ref[...] = jnp.maximum(acc + b_ref[...], 0.0)


def _tc_sage(x, partials, npart, deg_arr, w_self, w_neigh, b):
    out_dim = w_self.shape[1]
    return pl.pallas_call(
        _tc_sage_body,
        grid=(N_BLKS,),
        in_specs=[
            pl.BlockSpec((ROW_BLK, DIM), lambda i: (i, 0)),
            pl.BlockSpec((npart, ROW_BLK, DIM), lambda i: (0, i, 0)),
            pl.BlockSpec((1, ROW_BLK, DIM), lambda i: (1, i, 0)),
            pl.BlockSpec((DIM, out_dim), lambda i: (0, 0)),
            pl.BlockSpec((DIM, out_dim), lambda i: (0, 0)),
            pl.BlockSpec((1, out_dim), lambda i: (0, 0)),
        ],
        out_specs=pl.BlockSpec((ROW_BLK, out_dim), lambda i: (i, 0)),
        out_shape=jax.ShapeDtypeStruct((N_NODES, out_dim), jnp.float32),
    )(x, partials, deg_arr, w_self, w_neigh, b)


def kernel(x, edge_index, Ws_self, Ws_neigh, bs, Wh_self, Wh_neigh, bh):
    src = edge_index[0].astype(jnp.int32).reshape(NSLICE, NSUPER, SUBCH, CHUNK)
    dst = edge_index[1].astype(jnp.int32).reshape(NSLICE, NSUPER, SUBCH, CHUNK)
    zeros_nd = jnp.zeros((NPAD, DIM), jnp.float32)
    ones_rows = jnp.ones((CHUNK, DIM), jnp.float32)

    out1 = _sc_layer1(x, src, dst, zeros_nd, ones_rows)
    shared = _tc_sage(x, out1, 1, out1, Ws_self, Ws_neigh, bs.reshape(1, DIM))

    p2 = _sc_layer2(shared, src, dst, zeros_nd)
    w_self_cat = jnp.transpose(Wh_self, (1, 0, 2)).reshape(DIM, NUM_HEADS * DIM)
    w_neigh_cat = jnp.transpose(Wh_neigh, (1, 0, 2)).reshape(DIM, NUM_HEADS * DIM)
    b_cat = bh.reshape(1, NUM_HEADS * DIM)
    heads = _tc_sage(shared, p2, 2, out1, w_self_cat, w_neigh_cat, b_cat)
    return heads.reshape(N_NODES, NUM_HEADS, DIM)


# speedup vs baseline: 3.1614x; 1.4206x over previous
import jax
import jax.numpy as jnp
from jax import lax
from jax.experimental import pallas as pl
from jax.experimental.pallas import tpu as pltpu
from jax.experimental.pallas import tpu_sc as plsc

N_NODES = 10000
NPAD = 10240
DIM = 128
N_EDGES = 320000
NUM_HEADS = 8

NC = 2
NS = 16
NW = NC * NS
CHUNK = 80
NSLICE = NW
SLICE_E = N_EDGES // NSLICE
NSUPER = 5
SUBCH = SLICE_E // (NSUPER * CHUNK)
ROWS_PER_TILE = NPAD // NS


NBUF = 4


def _gather_scatter_ring(feat_hbm, src_v, dst_v, acc_sh, bufs, sems):
    for j in range(NBUF - 1):
        pltpu.async_copy(feat_hbm.at[src_v.at[j]], bufs[j], sems[j])

    def chunk_body(c, carry):
        for par in range(NBUF):
            cc = c * NBUF + par
            pltpu.make_async_copy(feat_hbm.at[src_v.at[cc]],
                                  bufs[par], sems[par]).wait()

            @pl.when(cc + NBUF - 1 < SUBCH)
            def _():
                nxt = (par + NBUF - 1) % NBUF
                pltpu.async_copy(feat_hbm.at[src_v.at[cc + NBUF - 1]],
                                 bufs[nxt], sems[nxt])

            pltpu.sync_copy(bufs[par], acc_sh.at[dst_v.at[cc]], add=True)
        return carry

    lax.fori_loop(0, SUBCH // NBUF, chunk_body, 0)
    for cc in range((SUBCH // NBUF) * NBUF, SUBCH):
        par = cc % NBUF
        pltpu.make_async_copy(feat_hbm.at[src_v.at[cc]],
                              bufs[par], sems[par]).wait()
        pltpu.sync_copy(bufs[par], acc_sh.at[dst_v.at[cc]], add=True)


def _sc_layer1(feat, src3, dst3, zeros_nd, ones_rows):
    def body(feat_hbm, src_hbm, dst_hbm, zeros_hbm, ones_hbm, out_hbm,
             src_v, dst_v, b0, b1, b2, b3, acc_sh, s0, s1, s2, s3):
        bufs = (b0, b1, b2, b3)
        sems = (s0, s1, s2, s3)
        rows_v = b0
        cid = lax.axis_index("c")
        sid = lax.axis_index("s")
        row0 = sid * ROWS_PER_TILE
        pltpu.sync_copy(zeros_hbm.at[pl.ds(row0, ROWS_PER_TILE)],
                        acc_sh.at[pl.ds(row0, ROWS_PER_TILE)])

        @pl.when(cid == 1)
        def _():
            pltpu.sync_copy(ones_hbm, rows_v)

        plsc.subcore_barrier()

        @pl.when(cid == 0)
        def _():
            def run_slice(t, carry):
                w = sid * 2 + t

                def super_body(s, carry1):
                    pltpu.sync_copy(dst_hbm.at[w, s], dst_v)
                    pltpu.sync_copy(src_hbm.at[w, s], src_v)
                    _gather_scatter_ring(feat_hbm, src_v, dst_v, acc_sh,
                                         bufs, sems)
                    return carry1

                lax.fori_loop(0, NSUPER, super_body, 0)
                return carry

            lax.fori_loop(0, 2, run_slice, 0)

        @pl.when(cid == 1)
        def _():
            def run_slice(t, carry):
                w = sid * 2 + t

                def super_body(s, carry1):
                    pltpu.sync_copy(dst_hbm.at[w, s], dst_v)

                    def chunk_body(c, carry2):
                        pltpu.sync_copy(rows_v, acc_sh.at[dst_v.at[c]],
                                        add=True)
                        return carry2

                    lax.fori_loop(0, SUBCH, chunk_body, 0)
                    return carry1

                lax.fori_loop(0, NSUPER, super_body, 0)
                return carry

            lax.fori_loop(0, 2, run_slice, 0)

        plsc.subcore_barrier()

        pltpu.sync_copy(acc_sh.at[pl.ds(row0, ROWS_PER_TILE)],
                        out_hbm.at[cid, pl.ds(row0, ROWS_PER_TILE)])

    mesh = plsc.VectorSubcoreMesh(core_axis_name="c", subcore_axis_name="s",
                                  num_cores=NC, num_subcores=NS)
    return pl.kernel(
        body,
        out_type=jax.ShapeDtypeStruct((NC, NPAD, DIM), jnp.float32),
        mesh=mesh,
        scratch_types=[
            pltpu.VMEM((SUBCH, CHUNK), jnp.int32),
            pltpu.VMEM((SUBCH, CHUNK), jnp.int32),
            pltpu.VMEM((CHUNK, DIM), jnp.float32),
            pltpu.VMEM((CHUNK, DIM), jnp.float32),
            pltpu.VMEM((CHUNK, DIM), jnp.float32),
            pltpu.VMEM((CHUNK, DIM), jnp.float32),
            pltpu.VMEM_SHARED((NPAD, DIM), jnp.float32),
            pltpu.SemaphoreType.DMA,
            pltpu.SemaphoreType.DMA,
            pltpu.SemaphoreType.DMA,
            pltpu.SemaphoreType.DMA,
        ],
        name="sc_sum_and_deg",
    )(feat, src3, dst3, zeros_nd, ones_rows)


def _sc_layer2(feat, src3, dst3, zeros_nd):
    def body(feat_hbm, src_hbm, dst_hbm, zeros_hbm, out_hbm,
             src_v, dst_v, b0, b1, b2, b3, acc_sh, s0, s1, s2, s3):
        bufs = (b0, b1, b2, b3)
        sems = (s0, s1, s2, s3)
        cid = lax.axis_index("c")
        sid = lax.axis_index("s")
        wid = cid * NS + sid
        row0 = sid * ROWS_PER_TILE
        pltpu.sync_copy(zeros_hbm.at[pl.ds(row0, ROWS_PER_TILE)],
                        acc_sh.at[pl.ds(row0, ROWS_PER_TILE)])
        plsc.subcore_barrier()

        def super_body(s, carry):
            pltpu.sync_copy(src_hbm.at[wid, s], src_v)
            pltpu.sync_copy(dst_hbm.at[wid, s], dst_v)
            _gather_scatter_ring(feat_hbm, src_v, dst_v, acc_sh,
                                 bufs, sems)
            return carry

        lax.fori_loop(0, NSUPER, super_body, 0)
        plsc.subcore_barrier()

        pltpu.sync_copy(acc_sh.at[pl.ds(row0, ROWS_PER_TILE)],
                        out_hbm.at[cid, pl.ds(row0, ROWS_PER_TILE)])

    mesh = plsc.VectorSubcoreMesh(core_axis_name="c", subcore_axis_name="s",
                                  num_cores=NC, num_subcores=NS)
    return pl.kernel(
        body,
        out_type=jax.ShapeDtypeStruct((NC, NPAD, DIM), jnp.float32),
        mesh=mesh,
        scratch_types=[
            pltpu.VMEM((SUBCH, CHUNK), jnp.int32),
            pltpu.VMEM((SUBCH, CHUNK), jnp.int32),
            pltpu.VMEM((CHUNK, DIM), jnp.float32),
            pltpu.VMEM((CHUNK, DIM), jnp.float32),
            pltpu.VMEM((CHUNK, DIM), jnp.float32),
            pltpu.VMEM((CHUNK, DIM), jnp.float32),
            pltpu.VMEM_SHARED((NPAD, DIM), jnp.float32),
            pltpu.SemaphoreType.DMA,
            pltpu.SemaphoreType.DMA,
            pltpu.SemaphoreType.DMA,
            pltpu.SemaphoreType.DMA,
        ],
        name="sc_seg_sum",
    )(feat, src3, dst3, zeros_nd)


ROW_BLK = 1000
N_BLKS = N_NODES // ROW_BLK


def _tc_sage_body(x_ref, p_ref, deg_ref, wself_ref, wneigh_ref, b_ref, out_ref):
    deg = jnp.maximum(deg_ref[0], 1.0)
    psum = p_ref[0]
    for c in range(1, p_ref.shape[0]):
        psum = psum + p_ref[c]
    agg = psum / deg
    acc = jnp.dot(x_ref[...], wself_ref[...], preferred_element_type=jnp.float32)
    acc += jnp.dot(agg, wneigh_ref[...], preferred_element_type=jnp.float32)
    out_ref[...] = jnp.maximum(acc + b_ref[...], 0.0)


def _tc_sage(x, partials, npart, deg_arr, w_self, w_neigh, b):
    out_dim = w_self.shape[1]
    return pl.pallas_call(
        _tc_sage_body,
        grid=(N_BLKS,),
        in_specs=[
            pl.BlockSpec((ROW_BLK, DIM), lambda i: (i, 0)),
            pl.BlockSpec((npart, ROW_BLK, DIM), lambda i: (0, i, 0)),
            pl.BlockSpec((1, ROW_BLK, DIM), lambda i: (1, i, 0)),
            pl.BlockSpec((DIM, out_dim), lambda i: (0, 0)),
            pl.BlockSpec((DIM, out_dim), lambda i: (0, 0)),
            pl.BlockSpec((1, out_dim), lambda i: (0, 0)),
        ],
        out_specs=pl.BlockSpec((ROW_BLK, out_dim), lambda i: (i, 0)),
        out_shape=jax.ShapeDtypeStruct((N_NODES, out_dim), jnp.float32),
    )(x, partials, deg_arr, w_self, w_neigh, b)


def kernel(x, edge_index, Ws_self, Ws_neigh, bs, Wh_self, Wh_neigh, bh):
    src = edge_index[0].astype(jnp.int32).reshape(NSLICE, NSUPER, SUBCH, CHUNK)
    dst = edge_index[1].astype(jnp.int32).reshape(NSLICE, NSUPER, SUBCH, CHUNK)
    zeros_nd = jnp.zeros((NPAD, DIM), jnp.float32)
    ones_rows = jnp.ones((CHUNK, DIM), jnp.float32)

    out1 = _sc_layer1(x, src, dst, zeros_nd, ones_rows)
    shared = _tc_sage(x, out1, 1, out1, Ws_self, Ws_neigh, bs.reshape(1, DIM))

    p2 = _sc_layer2(shared, src, dst, zeros_nd)
    w_self_cat = jnp.transpose(Wh_self, (1, 0, 2)).reshape(DIM, NUM_HEADS * DIM)
    w_neigh_cat = jnp.transpose(Wh_neigh, (1, 0, 2)).reshape(DIM, NUM_HEADS * DIM)
    b_cat = bh.reshape(1, NUM_HEADS * DIM)
    heads = _tc_sage(shared, p2, 2, out1, w_self_cat, w_neigh_cat, b_cat)
    return heads.reshape(N_NODES, NUM_HEADS, DIM)


# speedup vs baseline: 3.1971x; 1.0113x over previous
import jax
import jax.numpy as jnp
from jax import lax
from jax.experimental import pallas as pl
from jax.experimental.pallas import tpu as pltpu
from jax.experimental.pallas import tpu_sc as plsc

N_NODES = 10000
NPAD = 10240
DIM = 128
N_EDGES = 320000
NUM_HEADS = 8

NC = 2
NS = 16
NW = NC * NS
CHUNK = 80
NSLICE = NW
SLICE_E = N_EDGES // NSLICE
NSUPER = 5
SUBCH = SLICE_E // (NSUPER * CHUNK)
SUPER_LEN = SUBCH * CHUNK
ROWS_PER_TILE = NPAD // NS


NBUF = 4


def _gather_scatter_ring(feat_hbm, src_v, dst_v, acc_sh, bufs, sems):
    for j in range(NBUF - 1):
        pltpu.async_copy(feat_hbm.at[src_v.at[pl.ds(j * CHUNK, CHUNK)]], bufs[j], sems[j])

    def chunk_body(c, carry):
        for par in range(NBUF):
            cc = c * NBUF + par
            pltpu.make_async_copy(
                feat_hbm.at[src_v.at[pl.ds(cc * CHUNK, CHUNK)]],
                bufs[par], sems[par]).wait()

            @pl.when(cc + NBUF - 1 < SUBCH)
            def _():
                nxt = (par + NBUF - 1) % NBUF
                pltpu.async_copy(
                    feat_hbm.at[src_v.at[pl.ds((cc + NBUF - 1) * CHUNK, CHUNK)]],
                    bufs[nxt], sems[nxt])

            pltpu.sync_copy(bufs[par],
                            acc_sh.at[dst_v.at[pl.ds(cc * CHUNK, CHUNK)]],
                            add=True)
        return carry

    lax.fori_loop(0, SUBCH // NBUF, chunk_body, 0)
    for cc in range((SUBCH // NBUF) * NBUF, SUBCH):
        par = cc % NBUF
        pltpu.make_async_copy(
            feat_hbm.at[src_v.at[pl.ds(cc * CHUNK, CHUNK)]],
            bufs[par], sems[par]).wait()
        pltpu.sync_copy(bufs[par],
                        acc_sh.at[dst_v.at[pl.ds(cc * CHUNK, CHUNK)]],
                        add=True)


def _sc_layer1(feat, src3, dst3, zeros_nd, ones_rows):
    def body(feat_hbm, src_hbm, dst_hbm, zeros_hbm, ones_hbm, out_hbm,
             src_v, dst_v, b0, b1, b2, b3, acc_sh, s0, s1, s2, s3):
        bufs = (b0, b1, b2, b3)
        sems = (s0, s1, s2, s3)
        rows_v = b0
        cid = lax.axis_index("c")
        sid = lax.axis_index("s")
        row0 = sid * ROWS_PER_TILE
        pltpu.sync_copy(zeros_hbm.at[pl.ds(row0, ROWS_PER_TILE)],
                        acc_sh.at[pl.ds(row0, ROWS_PER_TILE)])

        @pl.when(cid == 1)
        def _():
            pltpu.sync_copy(ones_hbm, rows_v)

        plsc.subcore_barrier()

        @pl.when(cid == 0)
        def _():
            def run_slice(t, carry):
                w = sid * 2 + t

                def super_body(s, carry1):
                    base = w * SLICE_E + s * SUPER_LEN
                    pltpu.sync_copy(dst_hbm.at[pl.ds(base, SUPER_LEN)], dst_v)
                    pltpu.sync_copy(src_hbm.at[pl.ds(base, SUPER_LEN)], src_v)
                    _gather_scatter_ring(feat_hbm, src_v, dst_v, acc_sh,
                                         bufs, sems)
                    return carry1

                lax.fori_loop(0, NSUPER, super_body, 0)
                return carry

            lax.fori_loop(0, 2, run_slice, 0)

        @pl.when(cid == 1)
        def _():
            def run_slice(t, carry):
                w = sid * 2 + t

                def super_body(s, carry1):
                    base = w * SLICE_E + s * SUPER_LEN
                    pltpu.sync_copy(dst_hbm.at[pl.ds(base, SUPER_LEN)], dst_v)

                    def chunk_body(c, carry2):
                        pltpu.sync_copy(
                            rows_v,
                            acc_sh.at[dst_v.at[pl.ds(c * CHUNK, CHUNK)]],
                            add=True)
                        return carry2

                    lax.fori_loop(0, SUBCH, chunk_body, 0)
                    return carry1

                lax.fori_loop(0, NSUPER, super_body, 0)
                return carry

            lax.fori_loop(0, 2, run_slice, 0)

        plsc.subcore_barrier()

        pltpu.sync_copy(acc_sh.at[pl.ds(row0, ROWS_PER_TILE)],
                        out_hbm.at[cid, pl.ds(row0, ROWS_PER_TILE)])

    mesh = plsc.VectorSubcoreMesh(core_axis_name="c", subcore_axis_name="s",
                                  num_cores=NC, num_subcores=NS)
    return pl.kernel(
        body,
        out_type=jax.ShapeDtypeStruct((NC, NPAD, DIM), jnp.float32),
        mesh=mesh,
        scratch_types=[
            pltpu.VMEM((SUPER_LEN,), jnp.int32),
            pltpu.VMEM((SUPER_LEN,), jnp.int32),
            pltpu.VMEM((CHUNK, DIM), jnp.float32),
            pltpu.VMEM((CHUNK, DIM), jnp.float32),
            pltpu.VMEM((CHUNK, DIM), jnp.float32),
            pltpu.VMEM((CHUNK, DIM), jnp.float32),
            pltpu.VMEM_SHARED((NPAD, DIM), jnp.float32),
            pltpu.SemaphoreType.DMA,
            pltpu.SemaphoreType.DMA,
            pltpu.SemaphoreType.DMA,
            pltpu.SemaphoreType.DMA,
        ],
        name="sc_sum_and_deg",
    )(feat, src3, dst3, zeros_nd, ones_rows)


def _sc_layer2(feat, src3, dst3, zeros_nd):
    def body(feat_hbm, src_hbm, dst_hbm, zeros_hbm, out_hbm,
             src_v, dst_v, b0, b1, b2, b3, acc_sh, s0, s1, s2, s3):
        bufs = (b0, b1, b2, b3)
        sems = (s0, s1, s2, s3)
        cid = lax.axis_index("c")
        sid = lax.axis_index("s")
        wid = cid * NS + sid
        row0 = sid * ROWS_PER_TILE
        pltpu.sync_copy(zeros_hbm.at[pl.ds(row0, ROWS_PER_TILE)],
                        acc_sh.at[pl.ds(row0, ROWS_PER_TILE)])
        plsc.subcore_barrier()

        def super_body(s, carry):
            base = wid * SLICE_E + s * SUPER_LEN
            pltpu.sync_copy(src_hbm.at[pl.ds(base, SUPER_LEN)], src_v)
            pltpu.sync_copy(dst_hbm.at[pl.ds(base, SUPER_LEN)], dst_v)
            _gather_scatter_ring(feat_hbm, src_v, dst_v, acc_sh,
                                 bufs, sems)
            return carry

        lax.fori_loop(0, NSUPER, super_body, 0)
        plsc.subcore_barrier()

        pltpu.sync_copy(acc_sh.at[pl.ds(row0, ROWS_PER_TILE)],
                        out_hbm.at[cid, pl.ds(row0, ROWS_PER_TILE)])

    mesh = plsc.VectorSubcoreMesh(core_axis_name="c", subcore_axis_name="s",
                                  num_cores=NC, num_subcores=NS)
    return pl.kernel(
        body,
        out_type=jax.ShapeDtypeStruct((NC, NPAD, DIM), jnp.float32),
        mesh=mesh,
        scratch_types=[
            pltpu.VMEM((SUPER_LEN,), jnp.int32),
            pltpu.VMEM((SUPER_LEN,), jnp.int32),
            pltpu.VMEM((CHUNK, DIM), jnp.float32),
            pltpu.VMEM((CHUNK, DIM), jnp.float32),
            pltpu.VMEM((CHUNK, DIM), jnp.float32),
            pltpu.VMEM((CHUNK, DIM), jnp.float32),
            pltpu.VMEM_SHARED((NPAD, DIM), jnp.float32),
            pltpu.SemaphoreType.DMA,
            pltpu.SemaphoreType.DMA,
            pltpu.SemaphoreType.DMA,
            pltpu.SemaphoreType.DMA,
        ],
        name="sc_seg_sum",
    )(feat, src3, dst3, zeros_nd)


ROW_BLK = 1000
N_BLKS = N_NODES // ROW_BLK


def _tc_sage_body(x_ref, p_ref, deg_ref, wself_ref, wneigh_ref, b_ref, out_ref):
    deg = jnp.maximum(deg_ref[0], 1.0)
    psum = p_ref[0]
    for c in range(1, p_ref.shape[0]):
        psum = psum + p_ref[c]
    agg = psum / deg
    acc = jnp.dot(x_ref[...], wself_ref[...], preferred_element_type=jnp.float32)
    acc += jnp.dot(agg, wneigh_ref[...], preferred_element_type=jnp.float32)
    out_ref[...] = jnp.maximum(acc + b_ref[...], 0.0)


def _tc_sage(x, partials, npart, deg_arr, w_self, w_neigh, b):
    out_dim = w_self.shape[1]
    return pl.pallas_call(
        _tc_sage_body,
        grid=(N_BLKS,),
        in_specs=[
            pl.BlockSpec((ROW_BLK, DIM), lambda i: (i, 0)),
            pl.BlockSpec((npart, ROW_BLK, DIM), lambda i: (0, i, 0)),
            pl.BlockSpec((1, ROW_BLK, DIM), lambda i: (1, i, 0)),
            pl.BlockSpec((DIM, out_dim), lambda i: (0, 0)),
            pl.BlockSpec((DIM, out_dim), lambda i: (0, 0)),
            pl.BlockSpec((1, out_dim), lambda i: (0, 0)),
        ],
        out_specs=pl.BlockSpec((ROW_BLK, out_dim), lambda i: (i, 0)),
        out_shape=jax.ShapeDtypeStruct((N_NODES, out_dim), jnp.float32),
    )(x, partials, deg_arr, w_self, w_neigh, b)


def kernel(x, edge_index, Ws_self, Ws_neigh, bs, Wh_self, Wh_neigh, bh):
    src = edge_index[0].astype(jnp.int32)
    dst = edge_index[1].astype(jnp.int32)
    zeros_nd = jnp.zeros((NPAD, DIM), jnp.float32)
    ones_rows = jnp.ones((CHUNK, DIM), jnp.float32)

    out1 = _sc_layer1(x, src, dst, zeros_nd, ones_rows)
    shared = _tc_sage(x, out1, 1, out1, Ws_self, Ws_neigh, bs.reshape(1, DIM))

    p2 = _sc_layer2(shared, src, dst, zeros_nd)
    w_self_cat = jnp.transpose(Wh_self, (1, 0, 2)).reshape(DIM, NUM_HEADS * DIM)
    w_neigh_cat = jnp.transpose(Wh_neigh, (1, 0, 2)).reshape(DIM, NUM_HEADS * DIM)
    b_cat = bh.reshape(1, NUM_HEADS * DIM)
    heads = _tc_sage(shared, p2, 2, out1, w_self_cat, w_neigh_cat, b_cat)
    return heads.reshape(N_NODES, NUM_HEADS, DIM)
